# Initial kernel scaffold; baseline (speedup 1.0000x reference)
import jax
import jax.numpy as jnp
from jax.experimental import pallas as pl


def kernel(input_positions, weight):
    raise NotImplementedError("write your pallas kernel here")


# speedup vs baseline: 9.8884x; 9.8884x over previous
import functools

import jax
import jax.numpy as jnp
from jax import lax
from jax.experimental import pallas as pl
from jax.experimental.pallas import tpu as pltpu
from jax.experimental.pallas import tpu_sc as plsc

NC = 2
NS = 16
NW = NC * NS
L = 16
CH = 128
NBUF = 4


@functools.partial(jax.jit, static_argnums=(2, 3))
def _gather(weight, idx, nch, dim):
    bpw = nch * CH

    mesh = plsc.VectorSubcoreMesh(core_axis_name="c", subcore_axis_name="s")

    @functools.partial(
        pl.kernel,
        mesh=mesh,
        out_type=jax.ShapeDtypeStruct((NW * bpw, dim), jnp.float32),
        scratch_types=[
            pltpu.VMEM((nch, CH), jnp.int32),
            pltpu.VMEM((NBUF, CH, dim), jnp.float32),
            pltpu.SemaphoreType.DMA,
            pltpu.SemaphoreType.DMA,
        ],
    )
    def body(table_hbm, idx_hbm, out_hbm, idx_v, rows_v, gsem, ssem):
        c = lax.axis_index("c")
        s = lax.axis_index("s")
        wid = s * NC + c
        base = wid * bpw

        pltpu.sync_copy(idx_hbm.at[wid], idx_v)

        def group(g, carry):
            gcp = []
            for b in range(NBUF):
                j = g * NBUF + b
                for k in range(CH // L):
                    sl = pl.ds(k * L, L)
                    idx_v[j, sl] = idx_v[j, sl] + 1
                gcp.append(
                    pltpu.async_copy(table_hbm.at[idx_v.at[j]], rows_v.at[b], gsem)
                )
            scp = []
            for b in range(NBUF):
                j = g * NBUF + b
                gcp[b].wait()
                scp.append(
                    pltpu.async_copy(
                        rows_v.at[b], out_hbm.at[pl.ds(base + j * CH, CH)], ssem
                    )
                )
            for b in range(NBUF):
                scp[b].wait()
            return carry

        lax.fori_loop(0, nch // NBUF, group, 0)

    return body(weight, idx)


def kernel(input_positions, weight):
    bsz, slen = input_positions.shape
    dim = weight.shape[1]
    total = bsz * slen
    nch = total // (NW * CH)
    idx = input_positions.reshape(NW, nch, CH).astype(jnp.int32)
    out = _gather(weight, idx, nch, dim)
    return out.reshape(bsz, slen, dim)


# speedup vs baseline: 9.9531x; 1.0065x over previous
import functools

import jax
import jax.numpy as jnp
from jax import lax
from jax.experimental import pallas as pl
from jax.experimental.pallas import tpu as pltpu
from jax.experimental.pallas import tpu_sc as plsc

NC = 2
NS = 16
NW = NC * NS
L = 16
CH = 128
NBUF = 5
LAG = 2


@functools.partial(jax.jit, static_argnums=(2, 3))
def _gather(weight, idx, nch, dim):
    bpw = nch * CH
    ngrp = nch // NBUF

    mesh = plsc.VectorSubcoreMesh(core_axis_name="c", subcore_axis_name="s")

    @functools.partial(
        pl.kernel,
        mesh=mesh,
        out_type=jax.ShapeDtypeStruct((NW * bpw, dim), jnp.float32),
        scratch_types=[
            pltpu.VMEM((nch, CH), jnp.int32),
            pltpu.VMEM((NBUF, CH, dim), jnp.float32),
            pltpu.SemaphoreType.DMA,
            pltpu.SemaphoreType.DMA,
        ],
    )
    def body(table_hbm, idx_hbm, out_hbm, idx_v, rows_v, gsem, ssem):
        c = lax.axis_index("c")
        s = lax.axis_index("s")
        wid = s * NC + c
        base = wid * bpw

        pltpu.sync_copy(idx_hbm.at[wid], idx_v)

        def inc(j):
            for k in range(CH // L):
                sl = pl.ds(k * L, L)
                idx_v[j, sl] = idx_v[j, sl] + 1

        def start_gather(j, b):
            pltpu.async_copy(table_hbm.at[idx_v.at[j]], rows_v.at[b], gsem)

        def start_scatter(j, b):
            pltpu.async_copy(rows_v.at[b], out_hbm.at[pl.ds(base + j * CH, CH)], ssem)

        def wait_gather(b):
            pltpu.make_async_copy(table_hbm.at[pl.ds(0, CH)], rows_v.at[b], gsem).wait()

        def wait_scatter(b):
            pltpu.make_async_copy(rows_v.at[b], out_hbm.at[pl.ds(0, CH)], ssem).wait()

        for b in range(NBUF):
            inc(b)
            start_gather(b, b)
            if b >= LAG:
                wait_gather(b - LAG)
                start_scatter(b - LAG, b - LAG)

        def group(g, carry):
            for b in range(NBUF):
                j = g * NBUF + b
                inc(j)
                wait_scatter(b)
                start_gather(j, b)
                bp = (b - LAG) % NBUF
                wait_gather(bp)
                start_scatter(j - LAG, bp)
            return carry

        lax.fori_loop(1, ngrp, group, 0)

        for t in range(LAG):
            jj = nch - LAG + t
            b = jj % NBUF
            wait_gather(b)
            start_scatter(jj, b)
        for t in range(NBUF):
            wait_scatter(t)

    return body(weight, idx)


def kernel(input_positions, weight):
    bsz, slen = input_positions.shape
    dim = weight.shape[1]
    total = bsz * slen
    nch = total // (NW * CH)
    idx = input_positions.reshape(NW, nch, CH).astype(jnp.int32)
    out = _gather(weight, idx, nch, dim)
    return out.reshape(bsz, slen, dim)
